# baseline (speedup 1.0000x reference)
import functools

import jax
import jax.numpy as jnp
from jax import lax
from jax.experimental import pallas as pl
from jax.experimental.pallas import tpu as pltpu
from jax.experimental.pallas import tpu_sc as plsc

N = 10000
E = 160000
D = 256
H = 256
HALF = 128

NT = 16
CH = 80
NCHUNK = 125
EPAD = NT * NCHUNK * CH
RANGE = 2560
NPASS = 4
NPAD = RANGE * NPASS
ACC = RANGE + CH
ZPT = RANGE // NT
ZB = ZPT
WPT = RANGE // NT

BN = 400
NB = N // BN


_sc_mesh = plsc.VectorSubcoreMesh(core_axis_name="c", subcore_axis_name="s")


@functools.partial(
    pl.kernel,
    mesh=_sc_mesh,
    out_type=jax.ShapeDtypeStruct((2 * NPAD, HALF), jnp.float32),
    scratch_types=[
        pltpu.VMEM((NCHUNK, CH), jnp.int32),
        pltpu.VMEM((NCHUNK, CH), jnp.int32),
        pltpu.VMEM((1, CH), jnp.int32),
        pltpu.VMEM((1, CH), jnp.int32),
        pltpu.VMEM((CH, HALF), jnp.float32),
        pltpu.VMEM((CH, HALF), jnp.float32),
        pltpu.VMEM((ZB, HALF), jnp.float32),
        pltpu.VMEM((WPT, HALF), jnp.float32),
        pltpu.VMEM_SHARED((ACC, HALF), jnp.float32),
        pltpu.SemaphoreType.DMA,
        pltpu.SemaphoreType.DMA,
        pltpu.SemaphoreType.DMA,
        pltpu.SemaphoreType.DMA,
    ],
)
def _sc_agg(ztab, src, dst, out, srcv, dstv, dstadj0, dstadj1, rows0, rows1,
            zbuf, stagev, aggsh, sem0, sem1, ssem0, ssem1):
    c = lax.axis_index("c")
    s = lax.axis_index("s")

    pltpu.sync_copy(src.at[s], srcv)
    pltpu.sync_copy(dst.at[s], dstv)

    off = c * N

    def _adj(i, _):
        j = i // (CH // 16)
        k = i % (CH // 16)
        srcv[j, pl.ds(k * 16, 16)] = srcv[j, pl.ds(k * 16, 16)] + off
        return 0

    lax.fori_loop(0, NCHUNK * (CH // 16), _adj, 0)

    def _zero(i, _):
        r = i // (HALF // 16)
        k = i % (HALF // 16)
        zbuf[r, pl.ds(k * 16, 16)] = jnp.zeros((16,), jnp.float32)
        return 0

    lax.fori_loop(0, ZB * (HALF // 16), _zero, 0)

    iota = lax.iota(jnp.int32, 16)

    def _pass(p, _carry):
        lo = p * RANGE

        def _zcopy(t, _):
            pltpu.sync_copy(zbuf, aggsh.at[pl.ds(s * ZPT + t * ZB, ZB)])
            return 0

        lax.fori_loop(0, ZPT // ZB, _zcopy, 0)

        plsc.subcore_barrier()

        def _prep(buf, j):
            for k in range(CH // 16):
                d = dstv[j, pl.ds(k * 16, 16)] - lo
                valid = (d >= 0) & (d < RANGE)
                dummy = iota + (RANGE + k * 16)
                buf[0, pl.ds(k * 16, 16)] = jnp.where(valid, d, dummy)

        pltpu.async_copy(ztab.at[srcv.at[0]], rows0, sem0)
        pltpu.async_copy(ztab.at[srcv.at[1]], rows1, sem1)

        def _pair(t, _):
            j0 = 2 * t
            _prep(dstadj0, j0)
            pltpu.make_async_copy(ztab.at[srcv.at[j0]], rows0, sem0).wait()
            pltpu.async_copy(rows0, aggsh.at[dstadj0.at[0]], ssem0,
                             add=True)
            _prep(dstadj1, j0 + 1)
            pltpu.make_async_copy(ztab.at[srcv.at[j0 + 1]], rows1,
                                  sem1).wait()
            pltpu.async_copy(rows1, aggsh.at[dstadj1.at[0]], ssem1,
                             add=True)
            pltpu.make_async_copy(rows0, aggsh.at[dstadj0.at[0]],
                                  ssem0).wait()
            pltpu.async_copy(ztab.at[srcv.at[j0 + 2]], rows0, sem0)

            @pl.when(j0 + 3 < NCHUNK)
            def _():
                pltpu.make_async_copy(rows1, aggsh.at[dstadj1.at[0]],
                                      ssem1).wait()
                pltpu.async_copy(ztab.at[srcv.at[j0 + 3]], rows1, sem1)

            return 0

        lax.fori_loop(0, (NCHUNK - 1) // 2, _pair, 0)

        _prep(dstadj0, NCHUNK - 1)
        pltpu.make_async_copy(ztab.at[srcv.at[NCHUNK - 1]], rows0,
                              sem0).wait()
        pltpu.async_copy(rows0, aggsh.at[dstadj0.at[0]], ssem0, add=True)
        pltpu.make_async_copy(rows1, aggsh.at[dstadj1.at[0]], ssem1).wait()
        pltpu.make_async_copy(rows0, aggsh.at[dstadj0.at[0]], ssem0).wait()

        plsc.subcore_barrier()

        pltpu.sync_copy(aggsh.at[pl.ds(s * WPT, WPT)], stagev)
        pltpu.sync_copy(stagev, out.at[pl.ds(c * NPAD + lo + s * WPT, WPT)])

        plsc.subcore_barrier()
        return 0

    lax.fori_loop(0, NPASS, _pass, 0)


def _mlp_body(z_ref, agg_ref, w1_ref, b1_ref, w2_ref, b2_ref,
              hpre_ref, stats_ref):
    i = pl.program_id(0)
    agg = jnp.concatenate([agg_ref[0], agg_ref[1]], axis=1)
    h = z_ref[...] + agg
    h = lax.dot_general(h, w1_ref[...], (((1,), (1,)), ((), ())),
                        preferred_element_type=jnp.float32) + b1_ref[...]
    h = jnp.maximum(h, 0.0)
    h = lax.dot_general(h, w2_ref[...], (((1,), (1,)), ((), ())),
                        preferred_element_type=jnp.float32) + b2_ref[...]
    h = jnp.maximum(h, 0.0)
    hpre_ref[...] = h

    @pl.when(i == 0)
    def _():
        stats_ref[...] = jnp.zeros_like(stats_ref)

    stats_ref[0:1, :] = stats_ref[0:1, :] + jnp.sum(h, axis=0, keepdims=True)
    stats_ref[1:2, :] = stats_ref[1:2, :] + jnp.sum(h * h, axis=0,
                                                    keepdims=True)


def _mlp(z, agg, w1, b1, w2, b2):
    return pl.pallas_call(
        _mlp_body,
        grid=(NB,),
        in_specs=[
            pl.BlockSpec((BN, D), lambda i: (i, 0)),
            pl.BlockSpec((2, BN, HALF), lambda i: (0, i, 0)),
            pl.BlockSpec((H, D), lambda i: (0, 0)),
            pl.BlockSpec((1, H), lambda i: (0, 0)),
            pl.BlockSpec((H, H), lambda i: (0, 0)),
            pl.BlockSpec((1, H), lambda i: (0, 0)),
        ],
        out_specs=[
            pl.BlockSpec((BN, H), lambda i: (i, 0)),
            pl.BlockSpec((8, H), lambda i: (0, 0)),
        ],
        out_shape=[
            jax.ShapeDtypeStruct((N, H), jnp.float32),
            jax.ShapeDtypeStruct((8, H), jnp.float32),
        ],
    )(z, agg, w1, b1, w2, b2)


def _bn_body(hpre_ref, stats_ref, g_ref, b_ref, zout_ref, zst_ref):
    s1 = stats_ref[0:1, :]
    s2 = stats_ref[1:2, :]
    mean = s1 * (1.0 / N)
    var = s2 * (1.0 / N) - mean * mean
    scale = lax.rsqrt(var + 1e-5) * g_ref[...]
    zn = (hpre_ref[...] - mean) * scale + b_ref[...]
    zout_ref[...] = zn
    zst_ref[0] = zn[:, :HALF]
    zst_ref[1] = zn[:, HALF:]


def _bn(hpre, stats, gamma, beta):
    return pl.pallas_call(
        _bn_body,
        grid=(NB,),
        in_specs=[
            pl.BlockSpec((BN, H), lambda i: (i, 0)),
            pl.BlockSpec((8, H), lambda i: (0, 0)),
            pl.BlockSpec((1, H), lambda i: (0, 0)),
            pl.BlockSpec((1, H), lambda i: (0, 0)),
        ],
        out_specs=[
            pl.BlockSpec((BN, H), lambda i: (i, 0)),
            pl.BlockSpec((2, BN, HALF), lambda i: (0, i, 0)),
        ],
        out_shape=[
            jax.ShapeDtypeStruct((N, H), jnp.float32),
            jax.ShapeDtypeStruct((2, N, HALF), jnp.float32),
        ],
    )(hpre, stats, gamma, beta)


def kernel(x, edge_index, W1_0, b1_0, W2_0, b2_0, gamma_0, beta_0,
           W1_1, b1_1, W2_1, b2_1, gamma_1, beta_1):
    src = edge_index[0].reshape(NT, NCHUNK, CH)
    dst = edge_index[1].reshape(NT, NCHUNK, CH)

    xst = jnp.stack([x[:, :HALF], x[:, HALF:]]).reshape(2 * N, HALF)
    agg0 = _sc_agg(xst, src, dst).reshape(2, NPAD, HALF)
    hpre0, stats0 = _mlp(x, agg0, W1_0, b1_0.reshape(1, H),
                         W2_0, b2_0.reshape(1, H))
    z1, z1st = _bn(hpre0, stats0, gamma_0.reshape(1, H), beta_0.reshape(1, H))

    agg1 = _sc_agg(z1st.reshape(2 * N, HALF), src, dst).reshape(2, NPAD, HALF)
    hpre1, stats1 = _mlp(z1, agg1, W1_1, b1_1.reshape(1, H),
                         W2_1, b2_1.reshape(1, H))
    z2, _ = _bn(hpre1, stats1, gamma_1.reshape(1, H), beta_1.reshape(1, H))

    return jnp.concatenate([z1, z2], axis=1)


# speedup vs baseline: 1.2785x; 1.2785x over previous
import functools

import jax
import jax.numpy as jnp
from jax import lax
from jax.experimental import pallas as pl
from jax.experimental.pallas import tpu as pltpu
from jax.experimental.pallas import tpu_sc as plsc

N = 10000
E = 160000
D = 256
H = 256
HALF = 128

NT = 16
CH = 80
NCHUNK = 125
EPAD = NT * NCHUNK * CH
RANGE = 2560
NPASS = 4
NPAD = RANGE * NPASS
ACC = RANGE + CH
ZPT = RANGE // NT
ZB = ZPT
WPT = RANGE // NT

BN = 400
NB = N // BN


_sc_mesh = plsc.VectorSubcoreMesh(core_axis_name="c", subcore_axis_name="s")


@functools.partial(
    pl.kernel,
    mesh=_sc_mesh,
    out_type=jax.ShapeDtypeStruct((2 * NPAD, HALF), jnp.float32),
    scratch_types=[
        pltpu.VMEM((NCHUNK, CH), jnp.int32),
        pltpu.VMEM((NCHUNK, CH), jnp.int32),
        pltpu.VMEM((1, CH), jnp.int32),
        pltpu.VMEM((1, CH), jnp.int32),
        pltpu.VMEM((CH, HALF), jnp.float32),
        pltpu.VMEM((CH, HALF), jnp.float32),
        pltpu.VMEM((ZB, HALF), jnp.float32),
        pltpu.VMEM((WPT, HALF), jnp.float32),
        pltpu.VMEM_SHARED((ACC, HALF), jnp.float32),
        pltpu.SemaphoreType.DMA,
        pltpu.SemaphoreType.DMA,
    ],
)
def _sc_agg(ztab, src, dst, out, srcv, dstv, dstadj0, dstadj1, rows0, rows1,
            zbuf, stagev, aggsh, sem0, sem1):
    c = lax.axis_index("c")
    s = lax.axis_index("s")

    pltpu.sync_copy(src.at[s], srcv)
    pltpu.sync_copy(dst.at[s], dstv)

    off = c * N

    def _adj(i, _):
        j = i // (CH // 16)
        k = i % (CH // 16)
        srcv[j, pl.ds(k * 16, 16)] = srcv[j, pl.ds(k * 16, 16)] + off
        return 0

    lax.fori_loop(0, NCHUNK * (CH // 16), _adj, 0)

    def _zero(i, _):
        r = i // (HALF // 16)
        k = i % (HALF // 16)
        zbuf[r, pl.ds(k * 16, 16)] = jnp.zeros((16,), jnp.float32)
        return 0

    lax.fori_loop(0, ZB * (HALF // 16), _zero, 0)

    iota = lax.iota(jnp.int32, 16)

    def _pass(p, _carry):
        lo = p * RANGE

        def _zcopy(t, _):
            pltpu.sync_copy(zbuf, aggsh.at[pl.ds(s * ZPT + t * ZB, ZB)])
            return 0

        lax.fori_loop(0, ZPT // ZB, _zcopy, 0)

        plsc.subcore_barrier()

        def _prep(buf, j):
            for k in range(CH // 16):
                d = dstv[j, pl.ds(k * 16, 16)] - lo
                valid = (d >= 0) & (d < RANGE)
                dummy = iota + (RANGE + k * 16)
                buf[0, pl.ds(k * 16, 16)] = jnp.where(valid, d, dummy)

        pltpu.async_copy(ztab.at[srcv.at[0]], rows0, sem0)

        def _pair(t, _):
            j0 = 2 * t
            pltpu.async_copy(ztab.at[srcv.at[j0 + 1]], rows1, sem1)
            _prep(dstadj0, j0)
            pltpu.make_async_copy(ztab.at[srcv.at[j0]], rows0, sem0).wait()
            pltpu.sync_copy(rows0, aggsh.at[dstadj0.at[0]], add=True)
            pltpu.async_copy(ztab.at[srcv.at[j0 + 2]], rows0, sem0)
            _prep(dstadj1, j0 + 1)
            pltpu.make_async_copy(ztab.at[srcv.at[j0 + 1]], rows1,
                                  sem1).wait()
            pltpu.sync_copy(rows1, aggsh.at[dstadj1.at[0]], add=True)
            return 0

        lax.fori_loop(0, (NCHUNK - 1) // 2, _pair, 0)

        _prep(dstadj0, NCHUNK - 1)
        pltpu.make_async_copy(ztab.at[srcv.at[NCHUNK - 1]], rows0,
                              sem0).wait()
        pltpu.sync_copy(rows0, aggsh.at[dstadj0.at[0]], add=True)

        plsc.subcore_barrier()

        pltpu.sync_copy(aggsh.at[pl.ds(s * WPT, WPT)], stagev)
        pltpu.sync_copy(stagev, out.at[pl.ds(c * NPAD + lo + s * WPT, WPT)])

        plsc.subcore_barrier()
        return 0

    lax.fori_loop(0, NPASS, _pass, 0)


def _mlp_body(z_ref, agg_ref, w1_ref, b1_ref, w2_ref, b2_ref,
              hpre_ref, stats_ref):
    i = pl.program_id(0)
    agg = jnp.concatenate([agg_ref[0], agg_ref[1]], axis=1)
    h = z_ref[...] + agg
    h = lax.dot_general(h, w1_ref[...], (((1,), (1,)), ((), ())),
                        preferred_element_type=jnp.float32) + b1_ref[...]
    h = jnp.maximum(h, 0.0)
    h = lax.dot_general(h, w2_ref[...], (((1,), (1,)), ((), ())),
                        preferred_element_type=jnp.float32) + b2_ref[...]
    h = jnp.maximum(h, 0.0)
    hpre_ref[...] = h

    @pl.when(i == 0)
    def _():
        stats_ref[...] = jnp.zeros_like(stats_ref)

    stats_ref[0:1, :] = stats_ref[0:1, :] + jnp.sum(h, axis=0, keepdims=True)
    stats_ref[1:2, :] = stats_ref[1:2, :] + jnp.sum(h * h, axis=0,
                                                    keepdims=True)


def _mlp(z, agg, w1, b1, w2, b2):
    return pl.pallas_call(
        _mlp_body,
        grid=(NB,),
        in_specs=[
            pl.BlockSpec((BN, D), lambda i: (i, 0)),
            pl.BlockSpec((2, BN, HALF), lambda i: (0, i, 0)),
            pl.BlockSpec((H, D), lambda i: (0, 0)),
            pl.BlockSpec((1, H), lambda i: (0, 0)),
            pl.BlockSpec((H, H), lambda i: (0, 0)),
            pl.BlockSpec((1, H), lambda i: (0, 0)),
        ],
        out_specs=[
            pl.BlockSpec((BN, H), lambda i: (i, 0)),
            pl.BlockSpec((8, H), lambda i: (0, 0)),
        ],
        out_shape=[
            jax.ShapeDtypeStruct((N, H), jnp.float32),
            jax.ShapeDtypeStruct((8, H), jnp.float32),
        ],
    )(z, agg, w1, b1, w2, b2)


def _bn_body(hpre_ref, stats_ref, g_ref, b_ref, zout_ref, zst_ref):
    s1 = stats_ref[0:1, :]
    s2 = stats_ref[1:2, :]
    mean = s1 * (1.0 / N)
    var = s2 * (1.0 / N) - mean * mean
    scale = lax.rsqrt(var + 1e-5) * g_ref[...]
    zn = (hpre_ref[...] - mean) * scale + b_ref[...]
    zout_ref[...] = zn
    zst_ref[0] = zn[:, :HALF]
    zst_ref[1] = zn[:, HALF:]


def _bn(hpre, stats, gamma, beta):
    return pl.pallas_call(
        _bn_body,
        grid=(NB,),
        in_specs=[
            pl.BlockSpec((BN, H), lambda i: (i, 0)),
            pl.BlockSpec((8, H), lambda i: (0, 0)),
            pl.BlockSpec((1, H), lambda i: (0, 0)),
            pl.BlockSpec((1, H), lambda i: (0, 0)),
        ],
        out_specs=[
            pl.BlockSpec((BN, H), lambda i: (i, 0)),
            pl.BlockSpec((2, BN, HALF), lambda i: (0, i, 0)),
        ],
        out_shape=[
            jax.ShapeDtypeStruct((N, H), jnp.float32),
            jax.ShapeDtypeStruct((2, N, HALF), jnp.float32),
        ],
    )(hpre, stats, gamma, beta)


def kernel(x, edge_index, W1_0, b1_0, W2_0, b2_0, gamma_0, beta_0,
           W1_1, b1_1, W2_1, b2_1, gamma_1, beta_1):
    src = edge_index[0].reshape(NT, NCHUNK, CH)
    dst = edge_index[1].reshape(NT, NCHUNK, CH)

    xst = jnp.stack([x[:, :HALF], x[:, HALF:]]).reshape(2 * N, HALF)
    agg0 = _sc_agg(xst, src, dst).reshape(2, NPAD, HALF)
    hpre0, stats0 = _mlp(x, agg0, W1_0, b1_0.reshape(1, H),
                         W2_0, b2_0.reshape(1, H))
    z1, z1st = _bn(hpre0, stats0, gamma_0.reshape(1, H), beta_0.reshape(1, H))

    agg1 = _sc_agg(z1st.reshape(2 * N, HALF), src, dst).reshape(2, NPAD, HALF)
    hpre1, stats1 = _mlp(z1, agg1, W1_1, b1_1.reshape(1, H),
                         W2_1, b2_1.reshape(1, H))
    z2, _ = _bn(hpre1, stats1, gamma_1.reshape(1, H), beta_1.reshape(1, H))

    return jnp.concatenate([z1, z2], axis=1)
